# baseline (speedup 1.0000x reference)
import functools

import jax
import jax.numpy as jnp
from jax import lax
from jax.experimental import pallas as pl
from jax.experimental.pallas import tpu as pltpu
from jax.experimental.pallas import tpu_sc as plsc

N = 2048
H = 1024
F = 2048
E = 8
TOPK = 2

BLK = 256
G_MAX = N * TOPK // BLK + E - 1
P = G_MAX * BLK

NB = 8
TB = N // NB

NW = 32
TOK_PER = N // NW
CH = 32
WS_W = 128


def _router_body(x_ref, gw_ref, logits_ref, meta_ref, gidf_ref, w0x_ref,
                 w1x_ref, rc_scr, cs_scr):
    x = x_ref[...]
    logits = lax.dot_general(x, gw_ref[...], (((1,), (1,)), ((), ())),
                             preferred_element_type=jnp.float32)
    logits_ref[...] = logits

    idx8 = lax.broadcasted_iota(jnp.int32, (N, E), 1)
    l1 = jnp.max(logits, axis=1, keepdims=True)
    i1 = jnp.min(jnp.where(logits == l1, idx8, E), axis=1, keepdims=True)
    m1 = jnp.where(idx8 == i1, -jnp.inf, logits)
    l2 = jnp.max(m1, axis=1, keepdims=True)
    i2 = jnp.min(jnp.where(m1 == l2, idx8, E), axis=1, keepdims=True)
    wn1 = 1.0 / (1.0 + jnp.exp(l2 - l1))
    wn2 = 1.0 - wn1

    oh1 = (idx8 == i1).astype(jnp.float32)
    oh2 = (idx8 == i2).astype(jnp.float32)
    rc_scr[...] = oh1 + oh2

    rr = lax.broadcasted_iota(jnp.int32, (TB, TB), 0)
    cc = lax.broadcasted_iota(jnp.int32, (TB, TB), 1)
    tri = (cc < rr).astype(jnp.float32)

    def body(b, base):
        rc = rc_scr[pl.ds(b * TB, TB), :]
        part = lax.dot_general(tri, rc, (((1,), (0,)), ((), ())),
                               preferred_element_type=jnp.float32)
        cs_scr[pl.ds(b * TB, TB), :] = part + base
        return base + jnp.sum(rc, axis=0, keepdims=True)

    counts = lax.fori_loop(0, NB, body, jnp.zeros((1, E), jnp.float32))

    nblk = jnp.floor((counts + (BLK - 1)) * (1.0 / BLK))
    ii = lax.broadcasted_iota(jnp.int32, (E, E), 0)
    jj = lax.broadcasted_iota(jnp.int32, (E, E), 1)
    stri = (ii < jj).astype(jnp.float32)
    offb = lax.dot_general(nblk, stri, (((1,), (0,)), ((), ())),
                           preferred_element_type=jnp.float32)

    posmat = offb * float(BLK) + cs_scr[...]
    pos0 = jnp.sum(oh1 * posmat, axis=1, keepdims=True)
    pos1 = jnp.sum(oh2 * posmat, axis=1, keepdims=True)
    meta_ref[...] = jnp.concatenate(
        [pos0, pos1, wn1, wn2, jnp.zeros((N, 4), jnp.float32)], axis=1)
    ones16 = jnp.ones((1, WS_W), jnp.float32)
    w0x_ref[...] = wn1 * ones16
    w1x_ref[...] = wn2 * ones16

    ones = jnp.ones((1, 1), jnp.float32)
    nblk_c = lax.dot_general(nblk, ones, (((0,), (0,)), ((), ())))
    ii2 = lax.broadcasted_iota(jnp.int32, (E, E), 0)
    jj2 = lax.broadcasted_iota(jnp.int32, (E, E), 1)
    striL = (jj2 < ii2).astype(jnp.float32)
    offb_c = lax.dot_general(striL, nblk_c, (((1,), (0,)), ((), ())))
    gg = lax.broadcasted_iota(jnp.int32, (E, G_MAX + 8), 1).astype(
        jnp.float32)
    gid_row = jnp.sum((gg >= offb_c).astype(jnp.float32), axis=0,
                      keepdims=True) - 1.0
    used = jnp.sum(nblk, axis=1, keepdims=True)
    col = lax.broadcasted_iota(jnp.int32, (1, G_MAX + 8), 1)
    gidf_ref[...] = jnp.where(col == G_MAX, used, gid_row)


def _router_meta(x, gate_w):
    return pl.pallas_call(
        _router_body,
        out_shape=[
            jax.ShapeDtypeStruct((N, E), jnp.float32),
            jax.ShapeDtypeStruct((N, 8), jnp.float32),
            jax.ShapeDtypeStruct((1, G_MAX + 8), jnp.float32),
            jax.ShapeDtypeStruct((N, WS_W), jnp.float32),
            jax.ShapeDtypeStruct((N, WS_W), jnp.float32),
        ],
        scratch_shapes=[
            pltpu.VMEM((N, E), jnp.float32),
            pltpu.VMEM((N, E), jnp.float32),
        ],
    )(x, gate_w)


@functools.cache
def _make_sc_dispatch():
    mesh = plsc.VectorSubcoreMesh(core_axis_name="c", subcore_axis_name="s")

    @functools.partial(
        pl.kernel,
        mesh=mesh,
        out_type=[
            jax.ShapeDtypeStruct((P, H), jnp.float32),
            jax.ShapeDtypeStruct((P, WS_W), jnp.float32),
        ],
        scratch_types=[
            pltpu.VMEM((TOK_PER, H), jnp.float32),
            pltpu.VMEM((TOK_PER,), jnp.int32),
            pltpu.VMEM((TOK_PER,), jnp.int32),
            pltpu.VMEM((TOK_PER, WS_W), jnp.float32),
            pltpu.VMEM((TOK_PER, WS_W), jnp.float32),
            pltpu.SemaphoreType.DMA,
            pltpu.SemaphoreType.DMA,
            pltpu.SemaphoreType.DMA,
            pltpu.SemaphoreType.DMA,
        ],
    )
    def sc_dispatch(x_hbm, pos0_hbm, pos1_hbm, w0x_hbm, w1x_hbm, xs_hbm,
                    ws_hbm, xbuf, i0b, i1b, wb0, wb1, sem0, sem1, sem2,
                    sem3):
        wid = lax.axis_index("s") * 2 + lax.axis_index("c")
        b = wid * TOK_PER
        pltpu.sync_copy(x_hbm.at[pl.ds(b, TOK_PER)], xbuf)
        pltpu.sync_copy(pos0_hbm.at[pl.ds(b, TOK_PER)], i0b)
        pltpu.sync_copy(pos1_hbm.at[pl.ds(b, TOK_PER)], i1b)
        pltpu.sync_copy(w0x_hbm.at[pl.ds(b, TOK_PER)], wb0)
        pltpu.sync_copy(w1x_hbm.at[pl.ds(b, TOK_PER)], wb1)
        cp0 = pltpu.async_copy(xbuf, xs_hbm.at[i0b], sem0)
        cp1 = pltpu.async_copy(xbuf, xs_hbm.at[i1b], sem1)
        cp2 = pltpu.async_copy(wb0, ws_hbm.at[i0b], sem2)
        cp3 = pltpu.async_copy(wb1, ws_hbm.at[i1b], sem3)
        cp0.wait()
        cp1.wait()
        cp2.wait()
        cp3.wait()

    return sc_dispatch


def _sc_dispatch(*args):
    return _make_sc_dispatch()(*args)


def _mlp_body(gid_ref, xs_ref, wu_ref, wg_ref, wd_ref, bu_ref, bg_ref,
              bd_ref, ws_ref, ys_ref):
    g = pl.program_id(0)
    e = gid_ref[g]

    @pl.when(g < gid_ref[G_MAX])
    def _():
        x = xs_ref[...].astype(jnp.bfloat16)
        u = jnp.dot(x, wu_ref[0], preferred_element_type=jnp.float32)
        u = u + bu_ref[e, :][None, :]
        v = jnp.dot(x, wg_ref[0], preferred_element_type=jnp.float32)
        v = v + bg_ref[e, :][None, :]
        phi = (v * jax.nn.sigmoid(v) * u).astype(jnp.bfloat16)
        y = jnp.dot(phi, wd_ref[0], preferred_element_type=jnp.float32)
        y = y + bd_ref[e, :][None, :]
        ys_ref[...] = y * ws_ref[:, 0:1]


def _grouped_mlp(gid, xs, ws, w_up, w_gate, w_down, b_up, b_gate, b_down):
    grid_spec = pltpu.PrefetchScalarGridSpec(
        num_scalar_prefetch=1,
        grid=(G_MAX,),
        in_specs=[
            pl.BlockSpec((BLK, H), lambda g, gid: (g, 0)),
            pl.BlockSpec((1, H, F), lambda g, gid: (gid[g], 0, 0)),
            pl.BlockSpec((1, H, F), lambda g, gid: (gid[g], 0, 0)),
            pl.BlockSpec((1, F, H), lambda g, gid: (gid[g], 0, 0)),
            pl.BlockSpec((E, F), lambda g, gid: (0, 0)),
            pl.BlockSpec((E, F), lambda g, gid: (0, 0)),
            pl.BlockSpec((E, H), lambda g, gid: (0, 0)),
            pl.BlockSpec((BLK, WS_W), lambda g, gid: (g, 0)),
        ],
        out_specs=pl.BlockSpec((BLK, H), lambda g, gid: (g, 0)),
    )
    return pl.pallas_call(
        _mlp_body,
        grid_spec=grid_spec,
        out_shape=jax.ShapeDtypeStruct((P, H), jnp.float32),
        compiler_params=pltpu.CompilerParams(
            dimension_semantics=("arbitrary",),
        ),
    )(gid, xs, w_up, w_gate, w_down, b_up, b_gate, b_down, ws)


@functools.cache
def _make_sc_combine():
    mesh = plsc.VectorSubcoreMesh(core_axis_name="c", subcore_axis_name="s")
    DCH = 16
    NIT = TOK_PER // DCH

    @functools.partial(
        pl.kernel,
        mesh=mesh,
        out_type=jax.ShapeDtypeStruct((N, H), jnp.float32),
        scratch_types=[
            pltpu.VMEM((2, DCH, H), jnp.float32),
            pltpu.VMEM((2, DCH, H), jnp.float32),
            pltpu.VMEM((2, DCH, H), jnp.float32),
            pltpu.VMEM((NIT, DCH), jnp.int32),
            pltpu.VMEM((NIT, DCH), jnp.int32),
            pltpu.SemaphoreType.DMA,
            pltpu.SemaphoreType.DMA,
            pltpu.SemaphoreType.DMA,
            pltpu.SemaphoreType.DMA,
        ],
    )
    def sc_combine(ys_hbm, pos0_hbm, pos1_hbm, out_hbm, buf0, buf1, obuf,
                   i0b, i1b, gsem0, gsem1, osem0, osem1):
        wid = lax.axis_index("s") * 2 + lax.axis_index("c")
        base = wid * TOK_PER
        gsems = (gsem0, gsem1)
        osems = (osem0, osem1)
        for jj in range(NIT):
            pltpu.sync_copy(pos0_hbm.at[pl.ds(base + jj * DCH, DCH)],
                            i0b.at[jj])
            pltpu.sync_copy(pos1_hbm.at[pl.ds(base + jj * DCH, DCH)],
                            i1b.at[jj])

        def gathers(j):
            par = j % 2
            return (
                pltpu.async_copy(ys_hbm.at[i0b.at[j]], buf0.at[par],
                                 gsems[par]),
                pltpu.async_copy(ys_hbm.at[i1b.at[j]], buf1.at[par],
                                 gsems[par]),
            )

        gathers(0)
        for j in range(NIT):
            par = j % 2
            if j + 1 < NIT:
                gathers(j + 1)
            for cp in (
                pltpu.make_async_copy(ys_hbm.at[i0b.at[j]], buf0.at[par],
                                      gsems[par]),
                pltpu.make_async_copy(ys_hbm.at[i1b.at[j]], buf1.at[par],
                                      gsems[par]),
            ):
                cp.wait()
            if j >= 2:
                pltpu.make_async_copy(
                    obuf.at[par], out_hbm.at[pl.ds(base, DCH)],
                    osems[par]).wait()

            def addcol(cidx, carry):
                sl = pl.ds(cidx * 16, 16)
                for r in range(DCH):
                    obuf[par, r, sl] = buf0[par, r, sl] + buf1[par, r, sl]
                return carry

            lax.fori_loop(0, H // 16, addcol, 0)
            pltpu.async_copy(obuf.at[par],
                             out_hbm.at[pl.ds(base + j * DCH, DCH)],
                             osems[par])
        for j in (NIT - 2, NIT - 1):
            par = j % 2
            pltpu.make_async_copy(
                obuf.at[par], out_hbm.at[pl.ds(base + j * DCH, DCH)],
                osems[par]).wait()

    return sc_combine


def _sc_combine(*args):
    return _make_sc_combine()(*args)


@jax.jit
def kernel(hidden_states, gate_w, w_up, w_gate, w_down, b_up, b_gate, b_down):
    x = hidden_states.reshape(-1, H)
    logits, meta, gidf, w0x, w1x = _router_meta(x, gate_w)
    pos0 = meta[:, 0].astype(jnp.int32)
    pos1 = meta[:, 1].astype(jnp.int32)
    gid = gidf.reshape(G_MAX + 8).astype(jnp.int32)
    xs, ws = _sc_dispatch(x, pos0, pos1, w0x, w1x)
    ys = _grouped_mlp(gid, xs, ws, w_up.astype(jnp.bfloat16),
                      w_gate.astype(jnp.bfloat16),
                      w_down.astype(jnp.bfloat16), b_up, b_gate, b_down)
    out = _sc_combine(ys, pos0, pos1)
    return out.reshape(hidden_states.shape), logits


# speedup vs baseline: 1.3076x; 1.3076x over previous
import functools

import jax
import jax.numpy as jnp
from jax import lax
from jax.experimental import pallas as pl
from jax.experimental.pallas import tpu as pltpu
from jax.experimental.pallas import tpu_sc as plsc

N = 2048
H = 1024
F = 2048
E = 8
TOPK = 2

BLK = 256
G_MAX = N * TOPK // BLK + E - 1
P = G_MAX * BLK

NB = 8
TB = N // NB

NW = 32
TOK_PER = N // NW
CH = 32
WS_W = 128


def _router_body(x_ref, gw_ref, logits_ref, pos0_ref, pos1_ref, gidi_ref,
                 w0x_ref, w1x_ref, rc_scr, cs_scr):
    x = x_ref[...]
    logits = lax.dot_general(x, gw_ref[...], (((1,), (1,)), ((), ())),
                             preferred_element_type=jnp.float32)
    logits_ref[...] = logits

    idx8 = lax.broadcasted_iota(jnp.int32, (N, E), 1)
    l1 = jnp.max(logits, axis=1, keepdims=True)
    i1 = jnp.min(jnp.where(logits == l1, idx8, E), axis=1, keepdims=True)
    m1 = jnp.where(idx8 == i1, -jnp.inf, logits)
    l2 = jnp.max(m1, axis=1, keepdims=True)
    i2 = jnp.min(jnp.where(m1 == l2, idx8, E), axis=1, keepdims=True)
    wn1 = 1.0 / (1.0 + jnp.exp(l2 - l1))
    wn2 = 1.0 - wn1

    oh1 = (idx8 == i1).astype(jnp.float32)
    oh2 = (idx8 == i2).astype(jnp.float32)
    rc_scr[...] = oh1 + oh2

    rr = lax.broadcasted_iota(jnp.int32, (TB, TB), 0)
    cc = lax.broadcasted_iota(jnp.int32, (TB, TB), 1)
    tri = (cc < rr).astype(jnp.float32)

    def body(b, base):
        rc = rc_scr[pl.ds(b * TB, TB), :]
        part = lax.dot_general(tri, rc, (((1,), (0,)), ((), ())),
                               preferred_element_type=jnp.float32)
        cs_scr[pl.ds(b * TB, TB), :] = part + base
        return base + jnp.sum(rc, axis=0, keepdims=True)

    counts = lax.fori_loop(0, NB, body, jnp.zeros((1, E), jnp.float32))

    nblk = jnp.floor((counts + (BLK - 1)) * (1.0 / BLK))
    ii = lax.broadcasted_iota(jnp.int32, (E, E), 0)
    jj = lax.broadcasted_iota(jnp.int32, (E, E), 1)
    stri = (ii < jj).astype(jnp.float32)
    offb = lax.dot_general(nblk, stri, (((1,), (0,)), ((), ())),
                           preferred_element_type=jnp.float32)

    posmat = offb * float(BLK) + cs_scr[...]
    pos0 = jnp.sum(oh1 * posmat, axis=1, keepdims=True)
    pos1 = jnp.sum(oh2 * posmat, axis=1, keepdims=True)
    pos0_ref[...] = pos0.astype(jnp.int32)
    pos1_ref[...] = pos1.astype(jnp.int32)
    ones16 = jnp.ones((1, WS_W), jnp.float32)
    w0x_ref[...] = wn1 * ones16
    w1x_ref[...] = wn2 * ones16

    ones = jnp.ones((1, 1), jnp.float32)
    nblk_c = lax.dot_general(nblk, ones, (((0,), (0,)), ((), ())))
    ii2 = lax.broadcasted_iota(jnp.int32, (E, E), 0)
    jj2 = lax.broadcasted_iota(jnp.int32, (E, E), 1)
    striL = (jj2 < ii2).astype(jnp.float32)
    offb_c = lax.dot_general(striL, nblk_c, (((1,), (0,)), ((), ())))
    gg = lax.broadcasted_iota(jnp.int32, (E, G_MAX + 8), 1).astype(
        jnp.float32)
    gid_row = jnp.sum((gg >= offb_c).astype(jnp.float32), axis=0,
                      keepdims=True) - 1.0
    used = jnp.sum(nblk, axis=1, keepdims=True)
    col = lax.broadcasted_iota(jnp.int32, (1, G_MAX + 8), 1)
    gidi_ref[...] = jnp.where(col == G_MAX, used, gid_row).astype(jnp.int32)


def _router_meta(x, gate_w):
    return pl.pallas_call(
        _router_body,
        out_shape=[
            jax.ShapeDtypeStruct((N, E), jnp.float32),
            jax.ShapeDtypeStruct((N, 1), jnp.int32),
            jax.ShapeDtypeStruct((N, 1), jnp.int32),
            jax.ShapeDtypeStruct((1, G_MAX + 8), jnp.int32),
            jax.ShapeDtypeStruct((N, WS_W), jnp.float32),
            jax.ShapeDtypeStruct((N, WS_W), jnp.float32),
        ],
        scratch_shapes=[
            pltpu.VMEM((N, E), jnp.float32),
            pltpu.VMEM((N, E), jnp.float32),
        ],
    )(x, gate_w)


@functools.cache
def _make_sc_dispatch():
    mesh = plsc.VectorSubcoreMesh(core_axis_name="c", subcore_axis_name="s")

    @functools.partial(
        pl.kernel,
        mesh=mesh,
        out_type=[
            jax.ShapeDtypeStruct((P, H), jnp.float32),
            jax.ShapeDtypeStruct((P, WS_W), jnp.float32),
        ],
        scratch_types=[
            pltpu.VMEM((TOK_PER, H), jnp.float32),
            pltpu.VMEM((TOK_PER,), jnp.int32),
            pltpu.VMEM((TOK_PER,), jnp.int32),
            pltpu.VMEM((TOK_PER, WS_W), jnp.float32),
            pltpu.VMEM((TOK_PER, WS_W), jnp.float32),
            pltpu.SemaphoreType.DMA,
            pltpu.SemaphoreType.DMA,
            pltpu.SemaphoreType.DMA,
            pltpu.SemaphoreType.DMA,
        ],
    )
    def sc_dispatch(x_hbm, pos0_hbm, pos1_hbm, w0x_hbm, w1x_hbm, xs_hbm,
                    ws_hbm, xbuf, i0b, i1b, wb0, wb1, sem0, sem1, sem2,
                    sem3):
        wid = lax.axis_index("s") * 2 + lax.axis_index("c")
        b = wid * TOK_PER
        pltpu.sync_copy(x_hbm.at[pl.ds(b, TOK_PER)], xbuf)
        pltpu.sync_copy(pos0_hbm.at[pl.ds(b, TOK_PER)], i0b)
        pltpu.sync_copy(pos1_hbm.at[pl.ds(b, TOK_PER)], i1b)
        pltpu.sync_copy(w0x_hbm.at[pl.ds(b, TOK_PER)], wb0)
        pltpu.sync_copy(w1x_hbm.at[pl.ds(b, TOK_PER)], wb1)
        cp0 = pltpu.async_copy(xbuf, xs_hbm.at[i0b], sem0)
        cp1 = pltpu.async_copy(xbuf, xs_hbm.at[i1b], sem1)
        cp2 = pltpu.async_copy(wb0, ws_hbm.at[i0b], sem2)
        cp3 = pltpu.async_copy(wb1, ws_hbm.at[i1b], sem3)
        cp0.wait()
        cp1.wait()
        cp2.wait()
        cp3.wait()

    return sc_dispatch


def _sc_dispatch(*args):
    return _make_sc_dispatch()(*args)


def _mlp_body(gid_ref, xs_ref, wu_ref, wg_ref, wd_ref, bu_ref, bg_ref,
              bd_ref, ws_ref, ys_ref):
    g = pl.program_id(0)
    e = gid_ref[g]

    @pl.when(g < gid_ref[G_MAX])
    def _():
        x = xs_ref[...]
        u = jnp.dot(x, wu_ref[0], preferred_element_type=jnp.float32)
        u = u + bu_ref[e, :][None, :]
        v = jnp.dot(x, wg_ref[0], preferred_element_type=jnp.float32)
        v = v + bg_ref[e, :][None, :]
        phi = v * jax.nn.sigmoid(v) * u
        y = jnp.dot(phi, wd_ref[0], preferred_element_type=jnp.float32)
        y = y + bd_ref[e, :][None, :]
        ys_ref[...] = y * ws_ref[:, 0:1]


def _grouped_mlp(gid, xs, ws, w_up, w_gate, w_down, b_up, b_gate, b_down):
    grid_spec = pltpu.PrefetchScalarGridSpec(
        num_scalar_prefetch=1,
        grid=(G_MAX,),
        in_specs=[
            pl.BlockSpec((BLK, H), lambda g, gid: (g, 0)),
            pl.BlockSpec((1, H, F), lambda g, gid: (gid[g], 0, 0)),
            pl.BlockSpec((1, H, F), lambda g, gid: (gid[g], 0, 0)),
            pl.BlockSpec((1, F, H), lambda g, gid: (gid[g], 0, 0)),
            pl.BlockSpec((E, F), lambda g, gid: (0, 0)),
            pl.BlockSpec((E, F), lambda g, gid: (0, 0)),
            pl.BlockSpec((E, H), lambda g, gid: (0, 0)),
            pl.BlockSpec((BLK, WS_W), lambda g, gid: (g, 0)),
        ],
        out_specs=pl.BlockSpec((BLK, H), lambda g, gid: (g, 0)),
    )
    return pl.pallas_call(
        _mlp_body,
        grid_spec=grid_spec,
        out_shape=jax.ShapeDtypeStruct((P, H), jnp.float32),
        compiler_params=pltpu.CompilerParams(
            dimension_semantics=("arbitrary",),
        ),
    )(gid, xs, w_up, w_gate, w_down, b_up, b_gate, b_down, ws)


@functools.cache
def _make_sc_combine():
    mesh = plsc.VectorSubcoreMesh(core_axis_name="c", subcore_axis_name="s")
    DCH = 16
    NIT = TOK_PER // DCH

    @functools.partial(
        pl.kernel,
        mesh=mesh,
        out_type=jax.ShapeDtypeStruct((N, H), jnp.float32),
        scratch_types=[
            pltpu.VMEM((2, DCH, H), jnp.float32),
            pltpu.VMEM((2, DCH, H), jnp.float32),
            pltpu.VMEM((2, DCH, H), jnp.float32),
            pltpu.VMEM((NIT, DCH), jnp.int32),
            pltpu.VMEM((NIT, DCH), jnp.int32),
            pltpu.SemaphoreType.DMA,
            pltpu.SemaphoreType.DMA,
            pltpu.SemaphoreType.DMA,
            pltpu.SemaphoreType.DMA,
        ],
    )
    def sc_combine(ys_hbm, pos0_hbm, pos1_hbm, out_hbm, buf0, buf1, obuf,
                   i0b, i1b, gsem0, gsem1, osem0, osem1):
        wid = lax.axis_index("s") * 2 + lax.axis_index("c")
        base = wid * TOK_PER
        gsems = (gsem0, gsem1)
        osems = (osem0, osem1)
        for jj in range(NIT):
            pltpu.sync_copy(pos0_hbm.at[pl.ds(base + jj * DCH, DCH)],
                            i0b.at[jj])
            pltpu.sync_copy(pos1_hbm.at[pl.ds(base + jj * DCH, DCH)],
                            i1b.at[jj])

        def gathers(j):
            par = j % 2
            return (
                pltpu.async_copy(ys_hbm.at[i0b.at[j]], buf0.at[par],
                                 gsems[par]),
                pltpu.async_copy(ys_hbm.at[i1b.at[j]], buf1.at[par],
                                 gsems[par]),
            )

        gathers(0)
        for j in range(NIT):
            par = j % 2
            if j + 1 < NIT:
                gathers(j + 1)
            for cp in (
                pltpu.make_async_copy(ys_hbm.at[i0b.at[j]], buf0.at[par],
                                      gsems[par]),
                pltpu.make_async_copy(ys_hbm.at[i1b.at[j]], buf1.at[par],
                                      gsems[par]),
            ):
                cp.wait()
            if j >= 2:
                pltpu.make_async_copy(
                    obuf.at[par], out_hbm.at[pl.ds(base, DCH)],
                    osems[par]).wait()

            def addcol(cidx, carry):
                sl = pl.ds(cidx * 16, 16)
                for r in range(DCH):
                    obuf[par, r, sl] = buf0[par, r, sl] + buf1[par, r, sl]
                return carry

            lax.fori_loop(0, H // 16, addcol, 0)
            pltpu.async_copy(obuf.at[par],
                             out_hbm.at[pl.ds(base + j * DCH, DCH)],
                             osems[par])
        for j in (NIT - 2, NIT - 1):
            par = j % 2
            pltpu.make_async_copy(
                obuf.at[par], out_hbm.at[pl.ds(base + j * DCH, DCH)],
                osems[par]).wait()

    return sc_combine


def _sc_combine(*args):
    return _make_sc_combine()(*args)


@jax.jit
def kernel(hidden_states, gate_w, w_up, w_gate, w_down, b_up, b_gate, b_down):
    x = hidden_states.reshape(-1, H)
    logits, pos0o, pos1o, gido, w0x, w1x = _router_meta(x, gate_w)
    pos0 = pos0o.reshape(N)
    pos1 = pos1o.reshape(N)
    gid = gido.reshape(G_MAX + 8)
    xs, ws = _sc_dispatch(x, pos0, pos1, w0x, w1x)
    ys = _grouped_mlp(gid, xs, ws, w_up, w_gate, w_down, b_up, b_gate,
                      b_down)
    out = _sc_combine(ys, pos0, pos1)
    return out.reshape(hidden_states.shape), logits


# speedup vs baseline: 1.3220x; 1.0110x over previous
import functools

import jax
import jax.numpy as jnp
from jax import lax
from jax.experimental import pallas as pl
from jax.experimental.pallas import tpu as pltpu
from jax.experimental.pallas import tpu_sc as plsc

N = 2048
H = 1024
F = 2048
E = 8
TOPK = 2

BLK = 256
G_MAX = N * TOPK // BLK + E - 1
P = G_MAX * BLK

NB = 8
TB = N // NB

NW = 32
TOK_PER = N // NW
CH = 32
WS_W = 128


def _router_body(x_ref, gw_ref, logits_ref, meta_ref, gidi_ref, w0x_ref,
                 w1x_ref, rc_scr, cs_scr):
    x = x_ref[...]
    logits = lax.dot_general(x, gw_ref[...], (((1,), (1,)), ((), ())),
                             preferred_element_type=jnp.float32)
    logits_ref[...] = logits

    idx8 = lax.broadcasted_iota(jnp.int32, (N, E), 1)
    l1 = jnp.max(logits, axis=1, keepdims=True)
    i1 = jnp.min(jnp.where(logits == l1, idx8, E), axis=1, keepdims=True)
    m1 = jnp.where(idx8 == i1, -jnp.inf, logits)
    l2 = jnp.max(m1, axis=1, keepdims=True)
    i2 = jnp.min(jnp.where(m1 == l2, idx8, E), axis=1, keepdims=True)
    wn1 = 1.0 / (1.0 + jnp.exp(l2 - l1))
    wn2 = 1.0 - wn1

    oh1 = (idx8 == i1).astype(jnp.float32)
    oh2 = (idx8 == i2).astype(jnp.float32)
    rc_scr[...] = oh1 + oh2

    rr = lax.broadcasted_iota(jnp.int32, (TB, TB), 0)
    cc = lax.broadcasted_iota(jnp.int32, (TB, TB), 1)
    tri = (cc < rr).astype(jnp.float32)

    def body(b, base):
        rc = rc_scr[pl.ds(b * TB, TB), :]
        part = lax.dot_general(tri, rc, (((1,), (0,)), ((), ())),
                               preferred_element_type=jnp.float32)
        cs_scr[pl.ds(b * TB, TB), :] = part + base
        return base + jnp.sum(rc, axis=0, keepdims=True)

    counts = lax.fori_loop(0, NB, body, jnp.zeros((1, E), jnp.float32))

    nblk = jnp.floor((counts + (BLK - 1)) * (1.0 / BLK))
    ii = lax.broadcasted_iota(jnp.int32, (E, E), 0)
    jj = lax.broadcasted_iota(jnp.int32, (E, E), 1)
    stri = (ii < jj).astype(jnp.float32)
    offb = lax.dot_general(nblk, stri, (((1,), (0,)), ((), ())),
                           preferred_element_type=jnp.float32)

    posmat = offb * float(BLK) + cs_scr[...]
    pos0 = jnp.sum(oh1 * posmat, axis=1, keepdims=True)
    pos1 = jnp.sum(oh2 * posmat, axis=1, keepdims=True)
    meta_ref[...] = jnp.concatenate(
        [pos0, pos1, wn1, wn2, jnp.zeros((N, 4), jnp.float32)], axis=1)
    ones16 = jnp.ones((1, WS_W), jnp.float32)
    w0x_ref[...] = wn1 * ones16
    w1x_ref[...] = wn2 * ones16

    ones = jnp.ones((1, 1), jnp.float32)
    nblk_c = lax.dot_general(nblk, ones, (((0,), (0,)), ((), ())))
    ii2 = lax.broadcasted_iota(jnp.int32, (E, E), 0)
    jj2 = lax.broadcasted_iota(jnp.int32, (E, E), 1)
    striL = (jj2 < ii2).astype(jnp.float32)
    offb_c = lax.dot_general(striL, nblk_c, (((1,), (0,)), ((), ())))
    gg = lax.broadcasted_iota(jnp.int32, (E, G_MAX + 8), 1).astype(
        jnp.float32)
    gid_row = jnp.sum((gg >= offb_c).astype(jnp.float32), axis=0,
                      keepdims=True) - 1.0
    used = jnp.sum(nblk, axis=1, keepdims=True)
    col = lax.broadcasted_iota(jnp.int32, (1, G_MAX + 8), 1)
    gidi_ref[...] = jnp.where(col == G_MAX, used, gid_row).astype(jnp.int32)


def _router_meta(x, gate_w):
    return pl.pallas_call(
        _router_body,
        out_shape=[
            jax.ShapeDtypeStruct((N, E), jnp.float32),
            jax.ShapeDtypeStruct((N, 8), jnp.float32),
            jax.ShapeDtypeStruct((1, G_MAX + 8), jnp.int32),
            jax.ShapeDtypeStruct((N, WS_W), jnp.float32),
            jax.ShapeDtypeStruct((N, WS_W), jnp.float32),
        ],
        scratch_shapes=[
            pltpu.VMEM((N, E), jnp.float32),
            pltpu.VMEM((N, E), jnp.float32),
        ],
    )(x, gate_w)


@functools.cache
def _make_sc_dispatch():
    mesh = plsc.VectorSubcoreMesh(core_axis_name="c", subcore_axis_name="s")

    @functools.partial(
        pl.kernel,
        mesh=mesh,
        out_type=[
            jax.ShapeDtypeStruct((P, H), jnp.float32),
            jax.ShapeDtypeStruct((P, WS_W), jnp.float32),
        ],
        scratch_types=[
            pltpu.VMEM((TOK_PER, H), jnp.float32),
            pltpu.VMEM((TOK_PER,), jnp.int32),
            pltpu.VMEM((TOK_PER,), jnp.int32),
            pltpu.VMEM((TOK_PER, WS_W), jnp.float32),
            pltpu.VMEM((TOK_PER, WS_W), jnp.float32),
            pltpu.SemaphoreType.DMA,
            pltpu.SemaphoreType.DMA,
            pltpu.SemaphoreType.DMA,
            pltpu.SemaphoreType.DMA,
        ],
    )
    def sc_dispatch(x_hbm, pos0_hbm, pos1_hbm, w0x_hbm, w1x_hbm, xs_hbm,
                    ws_hbm, xbuf, i0b, i1b, wb0, wb1, sem0, sem1, sem2,
                    sem3):
        wid = lax.axis_index("s") * 2 + lax.axis_index("c")
        b = wid * TOK_PER
        pltpu.sync_copy(x_hbm.at[pl.ds(b, TOK_PER)], xbuf)
        pltpu.sync_copy(pos0_hbm.at[pl.ds(b, TOK_PER)], i0b)
        pltpu.sync_copy(pos1_hbm.at[pl.ds(b, TOK_PER)], i1b)
        pltpu.sync_copy(w0x_hbm.at[pl.ds(b, TOK_PER)], wb0)
        pltpu.sync_copy(w1x_hbm.at[pl.ds(b, TOK_PER)], wb1)
        cp0 = pltpu.async_copy(xbuf, xs_hbm.at[i0b], sem0)
        cp1 = pltpu.async_copy(xbuf, xs_hbm.at[i1b], sem1)
        cp2 = pltpu.async_copy(wb0, ws_hbm.at[i0b], sem2)
        cp3 = pltpu.async_copy(wb1, ws_hbm.at[i1b], sem3)
        cp0.wait()
        cp1.wait()
        cp2.wait()
        cp3.wait()

    return sc_dispatch


def _sc_dispatch(*args):
    return _make_sc_dispatch()(*args)


def _mlp_body(gid_ref, xs_ref, wu_ref, wg_ref, wd_ref, bu_ref, bg_ref,
              bd_ref, ws_ref, ys_ref):
    g = pl.program_id(0)
    e = gid_ref[g]

    @pl.when(g < gid_ref[G_MAX])
    def _():
        x = xs_ref[...]
        u = jnp.dot(x, wu_ref[0], preferred_element_type=jnp.float32)
        u = u + bu_ref[e, :][None, :]
        v = jnp.dot(x, wg_ref[0], preferred_element_type=jnp.float32)
        v = v + bg_ref[e, :][None, :]
        phi = v * jax.nn.sigmoid(v) * u
        y = jnp.dot(phi, wd_ref[0], preferred_element_type=jnp.float32)
        y = y + bd_ref[e, :][None, :]
        ys_ref[...] = y * ws_ref[:, 0:1]


def _grouped_mlp(gid, xs, ws, w_up, w_gate, w_down, b_up, b_gate, b_down):
    grid_spec = pltpu.PrefetchScalarGridSpec(
        num_scalar_prefetch=1,
        grid=(G_MAX,),
        in_specs=[
            pl.BlockSpec((BLK, H), lambda g, gid: (g, 0)),
            pl.BlockSpec((1, H, F), lambda g, gid: (gid[g], 0, 0)),
            pl.BlockSpec((1, H, F), lambda g, gid: (gid[g], 0, 0)),
            pl.BlockSpec((1, F, H), lambda g, gid: (gid[g], 0, 0)),
            pl.BlockSpec((E, F), lambda g, gid: (0, 0)),
            pl.BlockSpec((E, F), lambda g, gid: (0, 0)),
            pl.BlockSpec((E, H), lambda g, gid: (0, 0)),
            pl.BlockSpec((BLK, WS_W), lambda g, gid: (g, 0)),
        ],
        out_specs=pl.BlockSpec((BLK, H), lambda g, gid: (g, 0)),
    )
    return pl.pallas_call(
        _mlp_body,
        grid_spec=grid_spec,
        out_shape=jax.ShapeDtypeStruct((P, H), jnp.float32),
        compiler_params=pltpu.CompilerParams(
            dimension_semantics=("arbitrary",),
        ),
    )(gid, xs, w_up, w_gate, w_down, b_up, b_gate, b_down, ws)


@functools.cache
def _make_sc_combine():
    mesh = plsc.VectorSubcoreMesh(core_axis_name="c", subcore_axis_name="s")
    DCH = 16
    NIT = TOK_PER // DCH

    @functools.partial(
        pl.kernel,
        mesh=mesh,
        out_type=jax.ShapeDtypeStruct((N, H), jnp.float32),
        scratch_types=[
            pltpu.VMEM((2, DCH, H), jnp.float32),
            pltpu.VMEM((2, DCH, H), jnp.float32),
            pltpu.VMEM((2, DCH, H), jnp.float32),
            pltpu.VMEM((NIT, DCH), jnp.int32),
            pltpu.VMEM((NIT, DCH), jnp.int32),
            pltpu.SemaphoreType.DMA,
            pltpu.SemaphoreType.DMA,
            pltpu.SemaphoreType.DMA,
            pltpu.SemaphoreType.DMA,
        ],
    )
    def sc_combine(ys_hbm, pos0_hbm, pos1_hbm, out_hbm, buf0, buf1, obuf,
                   i0b, i1b, gsem0, gsem1, osem0, osem1):
        wid = lax.axis_index("s") * 2 + lax.axis_index("c")
        base = wid * TOK_PER
        gsems = (gsem0, gsem1)
        osems = (osem0, osem1)
        for jj in range(NIT):
            pltpu.sync_copy(pos0_hbm.at[pl.ds(base + jj * DCH, DCH)],
                            i0b.at[jj])
            pltpu.sync_copy(pos1_hbm.at[pl.ds(base + jj * DCH, DCH)],
                            i1b.at[jj])

        def gathers(j):
            par = j % 2
            return (
                pltpu.async_copy(ys_hbm.at[i0b.at[j]], buf0.at[par],
                                 gsems[par]),
                pltpu.async_copy(ys_hbm.at[i1b.at[j]], buf1.at[par],
                                 gsems[par]),
            )

        gathers(0)
        for j in range(NIT):
            par = j % 2
            if j + 1 < NIT:
                gathers(j + 1)
            for cp in (
                pltpu.make_async_copy(ys_hbm.at[i0b.at[j]], buf0.at[par],
                                      gsems[par]),
                pltpu.make_async_copy(ys_hbm.at[i1b.at[j]], buf1.at[par],
                                      gsems[par]),
            ):
                cp.wait()
            if j >= 2:
                pltpu.make_async_copy(
                    obuf.at[par], out_hbm.at[pl.ds(base, DCH)],
                    osems[par]).wait()

            def addcol(cidx, carry):
                sl = pl.ds(cidx * 16, 16)
                for r in range(DCH):
                    obuf[par, r, sl] = buf0[par, r, sl] + buf1[par, r, sl]
                return carry

            lax.fori_loop(0, H // 16, addcol, 0)
            pltpu.async_copy(obuf.at[par],
                             out_hbm.at[pl.ds(base + j * DCH, DCH)],
                             osems[par])
        for j in (NIT - 2, NIT - 1):
            par = j % 2
            pltpu.make_async_copy(
                obuf.at[par], out_hbm.at[pl.ds(base + j * DCH, DCH)],
                osems[par]).wait()

    return sc_combine


def _sc_combine(*args):
    return _make_sc_combine()(*args)


@jax.jit
def kernel(hidden_states, gate_w, w_up, w_gate, w_down, b_up, b_gate, b_down):
    x = hidden_states.reshape(-1, H)
    logits, meta, gido, w0x, w1x = _router_meta(x, gate_w)
    pos0 = meta[:, 0].astype(jnp.int32)
    pos1 = meta[:, 1].astype(jnp.int32)
    gid = gido.reshape(G_MAX + 8)
    xs, ws = _sc_dispatch(x, pos0, pos1, w0x, w1x)
    ys = _grouped_mlp(gid, xs, ws, w_up, w_gate, w_down, b_up, b_gate,
                      b_down)
    out = _sc_combine(ys, pos0, pos1)
    return out.reshape(hidden_states.shape), logits


# speedup vs baseline: 1.3238x; 1.0013x over previous
import functools

import jax
import jax.numpy as jnp
from jax import lax
from jax.experimental import pallas as pl
from jax.experimental.pallas import tpu as pltpu
from jax.experimental.pallas import tpu_sc as plsc

N = 2048
H = 1024
F = 2048
E = 8
TOPK = 2

BLK = 256
G_MAX = N * TOPK // BLK + E - 1
P = G_MAX * BLK

NB = 8
TB = N // NB

NW = 32
TOK_PER = N // NW
WS_W = 128


def _router_body(x_ref, gw_ref, logits_ref, meta_ref, gidi_ref, w0x_ref,
                 w1x_ref, rc_scr, cs_scr):
    x = x_ref[...]
    logits = lax.dot_general(x, gw_ref[...], (((1,), (1,)), ((), ())),
                             preferred_element_type=jnp.float32)
    logits_ref[...] = logits

    idx8 = lax.broadcasted_iota(jnp.int32, (N, E), 1)
    l1 = jnp.max(logits, axis=1, keepdims=True)
    i1 = jnp.min(jnp.where(logits == l1, idx8, E), axis=1, keepdims=True)
    m1 = jnp.where(idx8 == i1, -jnp.inf, logits)
    l2 = jnp.max(m1, axis=1, keepdims=True)
    i2 = jnp.min(jnp.where(m1 == l2, idx8, E), axis=1, keepdims=True)
    wn1 = 1.0 / (1.0 + jnp.exp(l2 - l1))
    wn2 = 1.0 - wn1

    oh1 = (idx8 == i1).astype(jnp.float32)
    oh2 = (idx8 == i2).astype(jnp.float32)
    rc_scr[...] = oh1 + oh2

    rr = lax.broadcasted_iota(jnp.int32, (TB, TB), 0)
    cc = lax.broadcasted_iota(jnp.int32, (TB, TB), 1)
    tri = (cc < rr).astype(jnp.float32)

    def body(b, base):
        rc = rc_scr[pl.ds(b * TB, TB), :]
        part = lax.dot_general(tri, rc, (((1,), (0,)), ((), ())),
                               preferred_element_type=jnp.float32)
        cs_scr[pl.ds(b * TB, TB), :] = part + base
        return base + jnp.sum(rc, axis=0, keepdims=True)

    counts = lax.fori_loop(0, NB, body, jnp.zeros((1, E), jnp.float32))

    nblk = jnp.floor((counts + (BLK - 1)) * (1.0 / BLK))
    ii = lax.broadcasted_iota(jnp.int32, (E, E), 0)
    jj = lax.broadcasted_iota(jnp.int32, (E, E), 1)
    stri = (ii < jj).astype(jnp.float32)
    offb = lax.dot_general(nblk, stri, (((1,), (0,)), ((), ())),
                           preferred_element_type=jnp.float32)

    posmat = offb * float(BLK) + cs_scr[...]
    pos0 = jnp.sum(oh1 * posmat, axis=1, keepdims=True)
    pos1 = jnp.sum(oh2 * posmat, axis=1, keepdims=True)
    meta_ref[...] = jnp.concatenate(
        [pos0, pos1, wn1, wn2, jnp.zeros((N, 4), jnp.float32)], axis=1)
    ones16 = jnp.ones((1, WS_W), jnp.float32)
    w0x_ref[...] = wn1 * ones16
    w1x_ref[...] = wn2 * ones16

    ones = jnp.ones((1, 1), jnp.float32)
    nblk_c = lax.dot_general(nblk, ones, (((0,), (0,)), ((), ())))
    ii2 = lax.broadcasted_iota(jnp.int32, (E, E), 0)
    jj2 = lax.broadcasted_iota(jnp.int32, (E, E), 1)
    striL = (jj2 < ii2).astype(jnp.float32)
    offb_c = lax.dot_general(striL, nblk_c, (((1,), (0,)), ((), ())))
    gg = lax.broadcasted_iota(jnp.int32, (E, G_MAX + 8), 1).astype(
        jnp.float32)
    gid_row = jnp.sum((gg >= offb_c).astype(jnp.float32), axis=0,
                      keepdims=True) - 1.0
    used = jnp.sum(nblk, axis=1, keepdims=True)
    col = lax.broadcasted_iota(jnp.int32, (1, G_MAX + 8), 1)
    gidi_ref[...] = jnp.where(col == G_MAX, used, gid_row).astype(jnp.int32)


def _router_meta(x, gate_w):
    return pl.pallas_call(
        _router_body,
        out_shape=[
            jax.ShapeDtypeStruct((N, E), jnp.float32),
            jax.ShapeDtypeStruct((N, 8), jnp.float32),
            jax.ShapeDtypeStruct((1, G_MAX + 8), jnp.int32),
            jax.ShapeDtypeStruct((N, WS_W), jnp.float32),
            jax.ShapeDtypeStruct((N, WS_W), jnp.float32),
        ],
        scratch_shapes=[
            pltpu.VMEM((N, E), jnp.float32),
            pltpu.VMEM((N, E), jnp.float32),
        ],
    )(x, gate_w)


@functools.cache
def _make_sc_dispatch():
    mesh = plsc.VectorSubcoreMesh(core_axis_name="c", subcore_axis_name="s")

    @functools.partial(
        pl.kernel,
        mesh=mesh,
        out_type=[
            jax.ShapeDtypeStruct((P, H), jnp.float32),
            jax.ShapeDtypeStruct((P, WS_W), jnp.float32),
        ],
        scratch_types=[
            pltpu.VMEM((TOK_PER, H), jnp.float32),
            pltpu.VMEM((TOK_PER,), jnp.int32),
            pltpu.VMEM((TOK_PER,), jnp.int32),
            pltpu.VMEM((TOK_PER, WS_W), jnp.float32),
            pltpu.VMEM((TOK_PER, WS_W), jnp.float32),
            pltpu.SemaphoreType.DMA,
            pltpu.SemaphoreType.DMA,
            pltpu.SemaphoreType.DMA,
            pltpu.SemaphoreType.DMA,
        ],
    )
    def sc_dispatch(x_hbm, pos0_hbm, pos1_hbm, w0x_hbm, w1x_hbm, xs_hbm,
                    ws_hbm, xbuf, i0b, i1b, wb0, wb1, sem0, sem1, sem2,
                    sem3):
        wid = lax.axis_index("s") * 2 + lax.axis_index("c")
        b = wid * TOK_PER
        pltpu.sync_copy(x_hbm.at[pl.ds(b, TOK_PER)], xbuf)
        pltpu.sync_copy(pos0_hbm.at[pl.ds(b, TOK_PER)], i0b)
        pltpu.sync_copy(pos1_hbm.at[pl.ds(b, TOK_PER)], i1b)
        pltpu.sync_copy(w0x_hbm.at[pl.ds(b, TOK_PER)], wb0)
        pltpu.sync_copy(w1x_hbm.at[pl.ds(b, TOK_PER)], wb1)
        cp0 = pltpu.async_copy(xbuf, xs_hbm.at[i0b], sem0)
        cp1 = pltpu.async_copy(xbuf, xs_hbm.at[i1b], sem1)
        cp2 = pltpu.async_copy(wb0, ws_hbm.at[i0b], sem2)
        cp3 = pltpu.async_copy(wb1, ws_hbm.at[i1b], sem3)
        cp0.wait()
        cp1.wait()
        cp2.wait()
        cp3.wait()

    return sc_dispatch


def _sc_dispatch(*args):
    return _make_sc_dispatch()(*args)


def _mlp_body(gid_ref, xs_ref, wu_ref, wg_ref, wd_ref, bu_ref, bg_ref,
              bd_ref, ws_ref, ys_ref):
    g = pl.program_id(0)
    e = gid_ref[g]

    @pl.when(g < gid_ref[G_MAX])
    def _():
        x = xs_ref[...]
        u = jnp.dot(x, wu_ref[0], preferred_element_type=jnp.float32)
        u = u + bu_ref[e, :][None, :]
        v = jnp.dot(x, wg_ref[0], preferred_element_type=jnp.float32)
        v = v + bg_ref[e, :][None, :]
        phi = v * jax.nn.sigmoid(v) * u
        y = jnp.dot(phi, wd_ref[0], preferred_element_type=jnp.float32)
        y = y + bd_ref[e, :][None, :]
        ys_ref[...] = y * ws_ref[:, 0:1]


def _grouped_mlp(gid, xs, ws, w_up, w_gate, w_down, b_up, b_gate, b_down):
    grid_spec = pltpu.PrefetchScalarGridSpec(
        num_scalar_prefetch=1,
        grid=(G_MAX,),
        in_specs=[
            pl.BlockSpec((BLK, H), lambda g, gid: (g, 0)),
            pl.BlockSpec((1, H, F), lambda g, gid: (gid[g], 0, 0)),
            pl.BlockSpec((1, H, F), lambda g, gid: (gid[g], 0, 0)),
            pl.BlockSpec((1, F, H), lambda g, gid: (gid[g], 0, 0)),
            pl.BlockSpec((E, F), lambda g, gid: (0, 0)),
            pl.BlockSpec((E, F), lambda g, gid: (0, 0)),
            pl.BlockSpec((E, H), lambda g, gid: (0, 0)),
            pl.BlockSpec((BLK, WS_W), lambda g, gid: (g, 0)),
        ],
        out_specs=pl.BlockSpec((BLK, H), lambda g, gid: (g, 0)),
    )
    return pl.pallas_call(
        _mlp_body,
        grid_spec=grid_spec,
        out_shape=jax.ShapeDtypeStruct((P, H), jnp.float32),
        compiler_params=pltpu.CompilerParams(
            dimension_semantics=("arbitrary",),
        ),
    )(gid, xs, w_up, w_gate, w_down, b_up, b_gate, b_down, ws)


@functools.cache
def _make_sc_combine():
    mesh = plsc.VectorSubcoreMesh(core_axis_name="c", subcore_axis_name="s")
    DCH = 16
    NIT = TOK_PER // DCH

    @functools.partial(
        pl.kernel,
        mesh=mesh,
        out_type=jax.ShapeDtypeStruct((N, H), jnp.float32),
        scratch_types=[
            pltpu.VMEM((2, DCH, H), jnp.float32),
            pltpu.VMEM((2, DCH, H), jnp.float32),
            pltpu.VMEM((2, DCH, H), jnp.float32),
            pltpu.VMEM((NIT, DCH), jnp.int32),
            pltpu.VMEM((NIT, DCH), jnp.int32),
            pltpu.SemaphoreType.DMA,
            pltpu.SemaphoreType.DMA,
            pltpu.SemaphoreType.DMA,
            pltpu.SemaphoreType.DMA,
        ],
    )
    def sc_combine(ys_hbm, pos0_hbm, pos1_hbm, out_hbm, buf0, buf1, obuf,
                   i0b, i1b, gsem0, gsem1, osem0, osem1):
        wid = lax.axis_index("s") * 2 + lax.axis_index("c")
        base = wid * TOK_PER
        gsems = (gsem0, gsem1)
        osems = (osem0, osem1)
        for jj in range(NIT):
            pltpu.sync_copy(pos0_hbm.at[pl.ds(base + jj * DCH, DCH)],
                            i0b.at[jj])
            pltpu.sync_copy(pos1_hbm.at[pl.ds(base + jj * DCH, DCH)],
                            i1b.at[jj])

        def gathers(j):
            par = j % 2
            return (
                pltpu.async_copy(ys_hbm.at[i0b.at[j]], buf0.at[par],
                                 gsems[par]),
                pltpu.async_copy(ys_hbm.at[i1b.at[j]], buf1.at[par],
                                 gsems[par]),
            )

        gathers(0)
        for j in range(NIT):
            par = j % 2
            if j + 1 < NIT:
                gathers(j + 1)
            for cp in (
                pltpu.make_async_copy(ys_hbm.at[i0b.at[j]], buf0.at[par],
                                      gsems[par]),
                pltpu.make_async_copy(ys_hbm.at[i1b.at[j]], buf1.at[par],
                                      gsems[par]),
            ):
                cp.wait()
            if j >= 2:
                pltpu.make_async_copy(
                    obuf.at[par], out_hbm.at[pl.ds(base, DCH)],
                    osems[par]).wait()

            def addcol(cidx, carry):
                sl = pl.ds(cidx * 16, 16)
                for r in range(DCH):
                    obuf[par, r, sl] = buf0[par, r, sl] + buf1[par, r, sl]
                return carry

            lax.fori_loop(0, H // 16, addcol, 0)
            pltpu.async_copy(obuf.at[par],
                             out_hbm.at[pl.ds(base + j * DCH, DCH)],
                             osems[par])
        for j in (NIT - 2, NIT - 1):
            par = j % 2
            pltpu.make_async_copy(
                obuf.at[par], out_hbm.at[pl.ds(base + j * DCH, DCH)],
                osems[par]).wait()

    return sc_combine


def _sc_combine(*args):
    return _make_sc_combine()(*args)


@jax.jit
def kernel(hidden_states, gate_w, w_up, w_gate, w_down, b_up, b_gate, b_down):
    x = hidden_states.reshape(-1, H)
    logits, meta, gido, w0x, w1x = _router_meta(x, gate_w)
    pos0 = meta[:, 0].astype(jnp.int32)
    pos1 = meta[:, 1].astype(jnp.int32)
    gid = gido.reshape(G_MAX + 8)
    xs, ws = _sc_dispatch(x, pos0, pos1, w0x, w1x)
    ys = _grouped_mlp(gid, xs, ws, w_up, w_gate, w_down, b_up, b_gate,
                      b_down)
    out = _sc_combine(ys, pos0, pos1)
    return out.reshape(hidden_states.shape), logits
